# baseline (speedup 1.0000x reference)
import functools

import jax
import jax.numpy as jnp
from jax import lax
from jax.experimental import pallas as pl
from jax.experimental.pallas import tpu as pltpu
from jax.experimental.pallas import tpu_sc as plsc

_BATCH = 16384
_DIM = 16
_BLK = 128
_ROWS_PER_BLK = _BLK // _DIM
_NUM_CORES = 2
_NUM_SUBCORES = 16
_NW = _NUM_CORES * _NUM_SUBCORES
_BPW = _BATCH // _NW
_CHUNK = 256
_N_CHUNKS = _BPW // _CHUNK

_STUDENT_ROWS = 1000000
_SUBJECT_ROWS = 100000
_SUBJECT_PAD = 102400

_mesh = plsc.VectorSubcoreMesh(core_axis_name="c", subcore_axis_name="s")


@functools.partial(
    pl.kernel,
    out_type=jax.ShapeDtypeStruct((_BATCH,), jnp.float32),
    mesh=_mesh,
    scratch_types=[
        pltpu.VMEM((_BPW,), jnp.int32),
        pltpu.VMEM((_BPW,), jnp.int32),
        pltpu.VMEM((_CHUNK,), jnp.int32),
        pltpu.VMEM((_CHUNK,), jnp.int32),
        pltpu.VMEM((_CHUNK, _BLK), jnp.float32),
        pltpu.VMEM((_CHUNK, _BLK), jnp.float32),
        pltpu.VMEM((_BPW,), jnp.float32),
        pltpu.SemaphoreType.DMA,
    ],
    compiler_params=pltpu.CompilerParams(needs_layout_passes=False),
)
def _mf_kernel(s_idx_hbm, u_idx_hbm, s_tab_hbm, u_tab_hbm, out_hbm,
               s_idx_v, u_idx_v, s_blkidx_v, u_blkidx_v, s_blk_v, u_blk_v,
               out_v, sem):
    wid = lax.axis_index("s") * _NUM_CORES + lax.axis_index("c")
    base = wid * _BPW
    pltpu.sync_copy(s_idx_hbm.at[pl.ds(base, _BPW)], s_idx_v)
    pltpu.sync_copy(u_idx_hbm.at[pl.ds(base, _BPW)], u_idx_v)

    lane = lax.iota(jnp.int32, _DIM)

    for c in range(_N_CHUNKS):
        cb = c * _CHUNK

        def blk_body(k, carry):
            src = pl.ds(cb + k * _DIM, _DIM)
            dst = pl.ds(k * _DIM, _DIM)
            s_blkidx_v[dst] = s_idx_v[src] >> 3
            u_blkidx_v[dst] = u_idx_v[src] >> 3
            return carry

        lax.fori_loop(0, _CHUNK // _DIM, blk_body, 0)
        g1 = pltpu.async_copy(s_tab_hbm.at[s_blkidx_v], s_blk_v, sem)
        g2 = pltpu.async_copy(u_tab_hbm.at[u_blkidx_v], u_blk_v, sem)
        g1.wait()
        g2.wait()

        def group_body(g, carry):
            row = g * _DIM
            rows16 = row + lane
            s_off = (s_idx_v[pl.ds(cb + row, _DIM)] & 7) * _DIM
            u_off = (u_idx_v[pl.ds(cb + row, _DIM)] & 7) * _DIM
            acc = jnp.zeros((_DIM,), jnp.float32)
            for d in range(_DIM):
                rot = (lane + d) & (_DIM - 1)
                s_col = plsc.load_gather(s_blk_v, [rows16, s_off + rot])
                u_col = plsc.load_gather(u_blk_v, [rows16, u_off + rot])
                acc = acc + s_col * u_col
            out_v[pl.ds(cb + row, _DIM)] = acc
            return carry

        lax.fori_loop(0, _CHUNK // _DIM, group_body, 0)

    pltpu.sync_copy(out_v, out_hbm.at[pl.ds(base, _BPW)])


def kernel(student_idx, subject_idx, student_table, subject_table):
    s_tab = student_table.reshape(_STUDENT_ROWS * _DIM // _BLK, _BLK)
    u_tab = jnp.pad(
        subject_table, ((0, _SUBJECT_PAD - _SUBJECT_ROWS), (0, 0))
    ).reshape(_SUBJECT_PAD * _DIM // _BLK, _BLK)
    return _mf_kernel(student_idx, subject_idx, s_tab, u_tab)
